# baseline (speedup 1.0000x reference)
import functools

import jax
import jax.numpy as jnp
import numpy as np
from jax import lax
from jax.experimental import pallas as pl
from jax.experimental.pallas import tpu as pltpu
from jax.experimental.pallas import tpu_sc as plsc

N = 10000
D = 128
S = 5000
NPAD = 10240
RSQ = np.float32(0.1 * 0.1)
K = 64
QPW = 160
SPAD = 32 * QPW
CAP = 256
FAR = np.float32(1e9)

RSQM = np.float32(0.1 * 0.1 * 1.001)
NEG_INF = np.float32(-np.inf)
POS_INF = np.float32(np.inf)


def _g_body(x_ref, w_ref, b_ref, o_ref):
    o_ref[...] = (
        jnp.dot(x_ref[...], w_ref[...], preferred_element_type=jnp.float32)
        + b_ref[...]
    )


def _compute_g(x_pad, wx, b2):
    grid = NPAD // 512
    return pl.pallas_call(
        _g_body,
        grid=(grid,),
        in_specs=[
            pl.BlockSpec((512, D), lambda i: (i, 0)),
            pl.BlockSpec((D, D), lambda i: (0, 0)),
            pl.BlockSpec((1, D), lambda i: (0, 0)),
        ],
        out_specs=pl.BlockSpec((512, D), lambda i: (i, 0)),
        out_shape=jax.ShapeDtypeStruct((NPAD, D), jnp.float32),
    )(x_pad, wx, b2)


def _fps_body(px_ref, py_ref, pz_ref, idx_ref, sx_ref, sy_ref, sz_ref):
    px = px_ref[...]
    py = py_ref[...]
    pz = pz_ref[...]
    flat = (lax.broadcasted_iota(jnp.int32, (8, 1280), 0) * 1280
            + lax.broadcasted_iota(jnp.int32, (8, 1280), 1))
    zero = flat == 0
    x0 = jnp.sum(jnp.where(zero, px, 0.0))
    y0 = jnp.sum(jnp.where(zero, py, 0.0))
    z0 = jnp.sum(jnp.where(zero, pz, 0.0))

    idx_ref[0] = jnp.int32(0)
    sx_ref[0] = x0
    sy_ref[0] = y0
    sz_ref[0] = z0

    d0 = (px - x0) ** 2 + (py - y0) ** 2 + (pz - z0) ** 2
    dists = jnp.where(flat < N, d0, NEG_INF)

    def body(i, dists):
        m = jnp.max(dists)
        nxt = jnp.min(jnp.where(dists == m, flat, jnp.int32(2 ** 30)))
        eq = flat == nxt
        sx = jnp.sum(jnp.where(eq, px, 0.0))
        sy = jnp.sum(jnp.where(eq, py, 0.0))
        sz = jnp.sum(jnp.where(eq, pz, 0.0))
        idx_ref[i] = nxt
        sx_ref[i] = sx
        sy_ref[i] = sy
        sz_ref[i] = sz
        dn = (px - sx) ** 2 + (py - sy) ** 2 + (pz - sz) ** 2
        return jnp.minimum(dists, dn)

    lax.fori_loop(1, S, body, dists)

    def pad(i, _):
        idx_ref[i] = jnp.int32(0)
        sx_ref[i] = x0
        sy_ref[i] = y0
        sz_ref[i] = z0
        return 0

    lax.fori_loop(S, SPAD, pad, 0)


def _run_fps(px, py, pz):
    smem = pl.BlockSpec(memory_space=pltpu.SMEM)
    return pl.pallas_call(
        _fps_body,
        out_shape=[
            jax.ShapeDtypeStruct((SPAD,), jnp.int32),
            jax.ShapeDtypeStruct((SPAD,), jnp.float32),
            jax.ShapeDtypeStruct((SPAD,), jnp.float32),
            jax.ShapeDtypeStruct((SPAD,), jnp.float32),
        ],
        out_specs=[smem, smem, smem, smem],
    )(px, py, pz)


def _cmin_body(qx_ref, qy_ref, qz_ref, pxt_ref, pyt_ref, pzt_ref, o_ref):
    qx = qx_ref[...].reshape(256, 1)
    qy = qy_ref[...].reshape(256, 1)
    qz = qz_ref[...].reshape(256, 1)
    cm = jnp.full((256, 640), POS_INF, jnp.float32)
    for t in range(16):
        dx = qx - pxt_ref[t, :].reshape(1, 640)
        dy = qy - pyt_ref[t, :].reshape(1, 640)
        dz = qz - pzt_ref[t, :].reshape(1, 640)
        d2 = (dx * dx + dy * dy) + dz * dz
        cm = jnp.minimum(cm, d2)
    o_ref[...] = cm


def _compute_cmins(qx, qy, qz, pxt, pyt, pzt):
    grid = SPAD // 256
    vec = pl.BlockSpec((256,), lambda i: (i,))
    full = pl.BlockSpec((16, 640), lambda i: (0, 0))
    return pl.pallas_call(
        _cmin_body,
        grid=(grid,),
        in_specs=[vec, vec, vec, full, full, full],
        out_specs=pl.BlockSpec((256, 640), lambda i: (i, 0)),
        out_shape=jax.ShapeDtypeStruct((SPAD, 640), jnp.float32),
    )(qx, qy, qz, pxt, pyt, pzt)


def _sc_conv_body(px_hbm, py_hbm, pz_hbm, qx_hbm, qy_hbm, qz_hbm, g_hbm,
                  wp_hbm, cm_hbm, out_hbm,
                  px_v, py_v, pz_v, qx_v, qy_v, qz_v,
                  ci_v, cx_v, cy_v, cz_v,
                  t_v, cm_v, fi_v, grows_v, wp_v, outq_v, sem):
    nc = 2
    wid = lax.axis_index("s") * nc + lax.axis_index("c")
    pltpu.sync_copy(px_hbm, px_v.at[pl.ds(0, NPAD)])
    pltpu.sync_copy(py_hbm, py_v.at[pl.ds(0, NPAD)])
    pltpu.sync_copy(pz_hbm, pz_v.at[pl.ds(0, NPAD)])
    qbase = wid * QPW
    pltpu.sync_copy(qx_hbm.at[pl.ds(qbase, QPW)], qx_v.at[pl.ds(0, QPW)])
    pltpu.sync_copy(qy_hbm.at[pl.ds(qbase, QPW)], qy_v.at[pl.ds(0, QPW)])
    pltpu.sync_copy(qz_hbm.at[pl.ds(qbase, QPW)], qz_v.at[pl.ds(0, QPW)])
    pltpu.sync_copy(wp_hbm, wp_v)

    def per_query(k, _):
        qxs = qx_v[pl.ds(k, 16)][0]
        qys = qy_v[pl.ds(k, 16)][0]
        qzs = qz_v[pl.ds(k, 16)][0]
        pltpu.sync_copy(cm_hbm.at[pl.ds((qbase + k) * 640, 640)],
                        cm_v.at[pl.ds(0, 640)])

        def chunk_group(g, nk):
            flags = [cm_v[pl.ds(g * 16 + u, 16)][0] for u in range(16)]
            for u in range(16):
                c = g * 16 + u
                cmc = flags[u]
                nk = _dig_if(cmc, c, nk)
            return nk

        def _dig_if(cmc, c, nk):
            def dig(nk):
                base = c * 16
                dx = px_v[pl.ds(base, 16)] - qxs
                dy = py_v[pl.ds(base, 16)] - qys
                dz = pz_v[pl.ds(base, 16)] - qzs
                d2 = (dx * dx + dy * dy) + dz * dz
                t_v[pl.ds(0, 16)] = d2
                for t in range(16):
                    d2t = t_v[pl.ds(t, 16)][0]
                    j = base + t
                    rx = px_v[pl.ds(j, 16)][0] - qxs
                    ry = py_v[pl.ds(j, 16)][0] - qys
                    rz = pz_v[pl.ds(j, 16)][0] - qzs
                    ok = (d2t <= RSQ) & (nk < CAP)
                    tgt = jnp.where(ok, nk, jnp.int32(CAP + 8))
                    ci_v[pl.ds(tgt, 16)] = jnp.full((16,), j, jnp.int32)
                    cx_v[pl.ds(tgt, 16)] = jnp.full((16,), rx, jnp.float32)
                    cy_v[pl.ds(tgt, 16)] = jnp.full((16,), ry, jnp.float32)
                    cz_v[pl.ds(tgt, 16)] = jnp.full((16,), rz, jnp.float32)
                    nk = nk + ok.astype(jnp.int32)
                return nk

            return lax.cond(cmc <= RSQM, dig, lambda nk: nk, nk)

        nk = lax.fori_loop(0, NPAD // 256, chunk_group, jnp.int32(0))

        @pl.when(nk <= K)
        def _fill():
            c0i = ci_v[pl.ds(0, 16)][0]
            c0x = cx_v[pl.ds(0, 16)][0]
            c0y = cy_v[pl.ds(0, 16)][0]
            c0z = cz_v[pl.ds(0, 16)][0]

            def pad1(t, _):
                ci_v[pl.ds(t, 16)] = jnp.full((16,), c0i, jnp.int32)
                cx_v[pl.ds(t, 16)] = jnp.full((16,), c0x, jnp.float32)
                cy_v[pl.ds(t, 16)] = jnp.full((16,), c0y, jnp.float32)
                cz_v[pl.ds(t, 16)] = jnp.full((16,), c0z, jnp.float32)
                return 0

            lax.fori_loop(nk, K, pad1, 0)

        @pl.when(nk > K)
        def _prune():
            def remove_one(j, nk):
                def scanmax(t, st):
                    bd, bi = st
                    rx = cx_v[pl.ds(t, 16)][0]
                    ry = cy_v[pl.ds(t, 16)][0]
                    rz = cz_v[pl.ds(t, 16)][0]
                    d = (rx * rx + ry * ry) + rz * rz
                    better = (d > bd) | ((d == bd) & (t > bi))
                    return (jnp.where(better, d, bd),
                            jnp.where(better, t, bi))

                bd, bi = lax.fori_loop(0, nk, scanmax,
                                       (NEG_INF, jnp.int32(-1)))
                last = nk - 1
                li = ci_v[pl.ds(last, 16)][0]
                lx = cx_v[pl.ds(last, 16)][0]
                ly = cy_v[pl.ds(last, 16)][0]
                lz = cz_v[pl.ds(last, 16)][0]
                iota = lax.iota(jnp.int32, 16)
                lane0 = iota == 0
                vi = ci_v[pl.ds(bi, 16)]
                ci_v[pl.ds(bi, 16)] = jnp.where(lane0, jnp.full((16,), li), vi)
                vx = cx_v[pl.ds(bi, 16)]
                cx_v[pl.ds(bi, 16)] = jnp.where(lane0, jnp.full((16,), lx), vx)
                vy = cy_v[pl.ds(bi, 16)]
                cy_v[pl.ds(bi, 16)] = jnp.where(lane0, jnp.full((16,), ly), vy)
                vz = cz_v[pl.ds(bi, 16)]
                cz_v[pl.ds(bi, 16)] = jnp.where(lane0, jnp.full((16,), lz), vz)
                return nk - 1

            lax.fori_loop(0, nk - K, remove_one, nk)

        for v in range(K // 16):
            fi_v[pl.ds(v * 16, 16)] = ci_v[pl.ds(v * 16, 16)]
        pltpu.async_copy(g_hbm.at[fi_v], grows_v, sem).wait()

        wpx = [wp_v[0, pl.ds(c * 16, 16)] for c in range(8)]
        wpy = [wp_v[1, pl.ds(c * 16, 16)] for c in range(8)]
        wpz = [wp_v[2, pl.ds(c * 16, 16)] for c in range(8)]
        acc = [jnp.zeros((16,), jnp.float32) for _ in range(8)]
        for kp in range(K):
            rxs = cx_v[pl.ds(kp, 16)][0]
            rys = cy_v[pl.ds(kp, 16)][0]
            rzs = cz_v[pl.ds(kp, 16)][0]
            for c in range(8):
                gv = grows_v[kp, pl.ds(c * 16, 16)]
                h = gv + rxs * wpx[c] + rys * wpy[c] + rzs * wpz[c]
                acc[c] = jnp.maximum(acc[c], jnp.maximum(h, 0.0))
        obase = k * D
        for c in range(8):
            outq_v[pl.ds(obase + c * 16, 16)] = acc[c]
        return 0

    lax.fori_loop(0, QPW, per_query, 0)
    pltpu.sync_copy(outq_v, out_hbm.at[pl.ds(qbase * D, QPW * D)])


def _run_sc_conv(px, py, pz, qx, qy, qz, g, wp, cmins):
    mesh = plsc.VectorSubcoreMesh(core_axis_name="c", subcore_axis_name="s",
                                  num_cores=2, num_subcores=16)
    f = functools.partial(
        pl.kernel,
        out_type=jax.ShapeDtypeStruct((SPAD * D,), jnp.float32),
        mesh=mesh,
        scratch_types=[
            pltpu.VMEM((NPAD + 16,), jnp.float32),
            pltpu.VMEM((NPAD + 16,), jnp.float32),
            pltpu.VMEM((NPAD + 16,), jnp.float32),
            pltpu.VMEM((QPW + 16,), jnp.float32),
            pltpu.VMEM((QPW + 16,), jnp.float32),
            pltpu.VMEM((QPW + 16,), jnp.float32),
            pltpu.VMEM((CAP + 32,), jnp.int32),
            pltpu.VMEM((CAP + 32,), jnp.float32),
            pltpu.VMEM((CAP + 32,), jnp.float32),
            pltpu.VMEM((CAP + 32,), jnp.float32),
            pltpu.VMEM((96,), jnp.float32),
            pltpu.VMEM((640 + 16,), jnp.float32),
            pltpu.VMEM((K,), jnp.int32),
            pltpu.VMEM((K, D), jnp.float32),
            pltpu.VMEM((3, D), jnp.float32),
            pltpu.VMEM((QPW * D,), jnp.float32),
            pltpu.SemaphoreType.DMA,
        ],
    )(_sc_conv_body)
    return f(px, py, pz, qx, qy, qz, g, wp, cmins)


def kernel(x, pos, batch, W, b):
    wx = W[:D, :]
    wp = W[D:, :]
    b2 = b.reshape(1, D)
    x_pad = jnp.pad(x, ((0, NPAD - N), (0, 0)))
    g = _compute_g(x_pad, wx, b2)

    posp = jnp.pad(pos, ((0, NPAD - N), (0, 0)), constant_values=FAR)
    planes = posp.T.reshape(3, 8, 1280)
    idx_pad, selx, sely, selz = _run_fps(planes[0], planes[1], planes[2])

    pts = posp.reshape(640, 16, 3)
    cmins = _compute_cmins(selx, sely, selz,
                           pts[:, :, 0].T.copy(), pts[:, :, 1].T.copy(),
                           pts[:, :, 2].T.copy())

    out_flat = _run_sc_conv(
        planes[0].reshape(NPAD), planes[1].reshape(NPAD),
        planes[2].reshape(NPAD), selx, sely, selz, g, wp,
        cmins.reshape(SPAD * 640))

    out = out_flat.reshape(SPAD, D)[:S]
    idx = idx_pad[:S]
    pos_dst = jnp.stack([selx[:S], sely[:S], selz[:S]], axis=1)
    batch_dst = batch[idx]
    return (out, pos_dst, batch_dst)


# speedup vs baseline: 3.4398x; 3.4398x over previous
import functools

import jax
import jax.numpy as jnp
import numpy as np
from jax import lax
from jax.experimental import pallas as pl
from jax.experimental.pallas import tpu as pltpu
from jax.experimental.pallas import tpu_sc as plsc

N = 10000
D = 128
S = 5000
NPAD = 10240
RSQ = np.float32(0.1 * 0.1)
K = 64
QPW = 160
SPAD = 32 * QPW
CAP = 256
FAR = np.float32(1e9)

RSQM = np.float32(0.1 * 0.1 * 1.001)
NEG_INF = np.float32(-np.inf)
POS_INF = np.float32(np.inf)


def _g_body(x_ref, w_ref, b_ref, o_ref):
    o_ref[...] = (
        jnp.dot(x_ref[...], w_ref[...], preferred_element_type=jnp.float32)
        + b_ref[...]
    )


def _compute_g(x_pad, wx, b2):
    grid = NPAD // 512
    return pl.pallas_call(
        _g_body,
        grid=(grid,),
        in_specs=[
            pl.BlockSpec((512, D), lambda i: (i, 0)),
            pl.BlockSpec((D, D), lambda i: (0, 0)),
            pl.BlockSpec((1, D), lambda i: (0, 0)),
        ],
        out_specs=pl.BlockSpec((512, D), lambda i: (i, 0)),
        out_shape=jax.ShapeDtypeStruct((NPAD, D), jnp.float32),
    )(x_pad, wx, b2)


def _fps_body(px_ref, py_ref, pz_ref, idx_ref, sx_ref, sy_ref, sz_ref):
    px = px_ref[...]
    py = py_ref[...]
    pz = pz_ref[...]
    flat = (lax.broadcasted_iota(jnp.int32, (8, 1280), 0) * 1280
            + lax.broadcasted_iota(jnp.int32, (8, 1280), 1))
    zero = flat == 0
    x0 = jnp.sum(jnp.where(zero, px, 0.0))
    y0 = jnp.sum(jnp.where(zero, py, 0.0))
    z0 = jnp.sum(jnp.where(zero, pz, 0.0))

    idx_ref[0] = jnp.int32(0)
    sx_ref[0] = x0
    sy_ref[0] = y0
    sz_ref[0] = z0

    d0 = (px - x0) ** 2 + (py - y0) ** 2 + (pz - z0) ** 2
    dists = jnp.where(flat < N, d0, NEG_INF)

    def body(i, dists):
        m = jnp.max(dists)
        nxt = jnp.min(jnp.where(dists == m, flat, jnp.int32(2 ** 30)))
        eq = flat == nxt
        sx = jnp.sum(jnp.where(eq, px, 0.0))
        sy = jnp.sum(jnp.where(eq, py, 0.0))
        sz = jnp.sum(jnp.where(eq, pz, 0.0))
        idx_ref[i] = nxt
        sx_ref[i] = sx
        sy_ref[i] = sy
        sz_ref[i] = sz
        dn = (px - sx) ** 2 + (py - sy) ** 2 + (pz - sz) ** 2
        return jnp.minimum(dists, dn)

    lax.fori_loop(1, S, body, dists)

    def pad(i, _):
        idx_ref[i] = jnp.int32(0)
        sx_ref[i] = x0
        sy_ref[i] = y0
        sz_ref[i] = z0
        return 0

    lax.fori_loop(S, SPAD, pad, 0)


def _run_fps(px, py, pz):
    smem = pl.BlockSpec(memory_space=pltpu.SMEM)
    return pl.pallas_call(
        _fps_body,
        out_shape=[
            jax.ShapeDtypeStruct((SPAD,), jnp.int32),
            jax.ShapeDtypeStruct((SPAD,), jnp.float32),
            jax.ShapeDtypeStruct((SPAD,), jnp.float32),
            jax.ShapeDtypeStruct((SPAD,), jnp.float32),
        ],
        out_specs=[smem, smem, smem, smem],
    )(px, py, pz)


def _cmin_body(qx_ref, qy_ref, qz_ref, pxt_ref, pyt_ref, pzt_ref, o_ref):
    qx = qx_ref[...].reshape(256, 1)
    qy = qy_ref[...].reshape(256, 1)
    qz = qz_ref[...].reshape(256, 1)
    cm = jnp.full((256, 640), POS_INF, jnp.float32)
    for t in range(16):
        dx = qx - pxt_ref[t, :].reshape(1, 640)
        dy = qy - pyt_ref[t, :].reshape(1, 640)
        dz = qz - pzt_ref[t, :].reshape(1, 640)
        d2 = (dx * dx + dy * dy) + dz * dz
        cm = jnp.minimum(cm, d2)
    o_ref[...] = cm


def _compute_cmins(qx, qy, qz, pxt, pyt, pzt):
    grid = SPAD // 256
    vec = pl.BlockSpec((256,), lambda i: (i,))
    full = pl.BlockSpec((16, 640), lambda i: (0, 0))
    return pl.pallas_call(
        _cmin_body,
        grid=(grid,),
        in_specs=[vec, vec, vec, full, full, full],
        out_specs=pl.BlockSpec((256, 640), lambda i: (i, 0)),
        out_shape=jax.ShapeDtypeStruct((SPAD, 640), jnp.float32),
    )(qx, qy, qz, pxt, pyt, pzt)


def _sc_conv_body(px_hbm, py_hbm, pz_hbm, qx_hbm, qy_hbm, qz_hbm, g_hbm,
                  wp_hbm, cm_hbm, out_hbm,
                  px_v, py_v, pz_v, qx_v, qy_v, qz_v,
                  ci_v, cx_v, cy_v, cz_v,
                  t_v, cm_v, fi_v, grows_v, wp_v, outq_v, sem):
    nc = 2
    wid = lax.axis_index("s") * nc + lax.axis_index("c")
    pltpu.sync_copy(px_hbm, px_v.at[pl.ds(0, NPAD)])
    pltpu.sync_copy(py_hbm, py_v.at[pl.ds(0, NPAD)])
    pltpu.sync_copy(pz_hbm, pz_v.at[pl.ds(0, NPAD)])
    qbase = wid * QPW
    pltpu.sync_copy(qx_hbm.at[pl.ds(qbase, QPW)], qx_v.at[pl.ds(0, QPW)])
    pltpu.sync_copy(qy_hbm.at[pl.ds(qbase, QPW)], qy_v.at[pl.ds(0, QPW)])
    pltpu.sync_copy(qz_hbm.at[pl.ds(qbase, QPW)], qz_v.at[pl.ds(0, QPW)])
    pltpu.sync_copy(wp_hbm, wp_v)

    def per_query(k, _):
        qxs = qx_v[pl.ds(k, 16)][0]
        qys = qy_v[pl.ds(k, 16)][0]
        qzs = qz_v[pl.ds(k, 16)][0]
        pltpu.sync_copy(cm_hbm.at[pl.ds((qbase + k) * 640, 640)],
                        cm_v.at[pl.ds(0, 640)])

        def chunk(c, nk):
            cmc = cm_v[pl.ds(c, 16)][0]

            def dig(nk):
                base = c * 16
                dx = px_v[pl.ds(base, 16)] - qxs
                dy = py_v[pl.ds(base, 16)] - qys
                dz = pz_v[pl.ds(base, 16)] - qzs
                d2 = (dx * dx + dy * dy) + dz * dz
                t_v[pl.ds(0, 16)] = d2
                for t in range(16):
                    d2t = t_v[pl.ds(t, 16)][0]
                    j = base + t
                    rx = px_v[pl.ds(j, 16)][0] - qxs
                    ry = py_v[pl.ds(j, 16)][0] - qys
                    rz = pz_v[pl.ds(j, 16)][0] - qzs
                    ok = (d2t <= RSQ) & (nk < CAP)
                    tgt = jnp.where(ok, nk, jnp.int32(CAP + 8))
                    ci_v[pl.ds(tgt, 16)] = jnp.full((16,), j, jnp.int32)
                    cx_v[pl.ds(tgt, 16)] = jnp.full((16,), rx, jnp.float32)
                    cy_v[pl.ds(tgt, 16)] = jnp.full((16,), ry, jnp.float32)
                    cz_v[pl.ds(tgt, 16)] = jnp.full((16,), rz, jnp.float32)
                    nk = nk + ok.astype(jnp.int32)
                return nk

            return lax.cond(cmc <= RSQM, dig, lambda nk: nk, nk)

        nk = lax.fori_loop(0, NPAD // 16, chunk, jnp.int32(0))

        @pl.when(nk <= K)
        def _fill():
            c0i = ci_v[pl.ds(0, 16)][0]
            c0x = cx_v[pl.ds(0, 16)][0]
            c0y = cy_v[pl.ds(0, 16)][0]
            c0z = cz_v[pl.ds(0, 16)][0]

            def pad1(t, _):
                ci_v[pl.ds(t, 16)] = jnp.full((16,), c0i, jnp.int32)
                cx_v[pl.ds(t, 16)] = jnp.full((16,), c0x, jnp.float32)
                cy_v[pl.ds(t, 16)] = jnp.full((16,), c0y, jnp.float32)
                cz_v[pl.ds(t, 16)] = jnp.full((16,), c0z, jnp.float32)
                return 0

            lax.fori_loop(nk, K, pad1, 0)

        @pl.when(nk > K)
        def _prune():
            def remove_one(j, nk):
                def scanmax(t, st):
                    bd, bi = st
                    rx = cx_v[pl.ds(t, 16)][0]
                    ry = cy_v[pl.ds(t, 16)][0]
                    rz = cz_v[pl.ds(t, 16)][0]
                    d = (rx * rx + ry * ry) + rz * rz
                    better = (d > bd) | ((d == bd) & (t > bi))
                    return (jnp.where(better, d, bd),
                            jnp.where(better, t, bi))

                bd, bi = lax.fori_loop(0, nk, scanmax,
                                       (NEG_INF, jnp.int32(-1)))
                last = nk - 1
                li = ci_v[pl.ds(last, 16)][0]
                lx = cx_v[pl.ds(last, 16)][0]
                ly = cy_v[pl.ds(last, 16)][0]
                lz = cz_v[pl.ds(last, 16)][0]
                iota = lax.iota(jnp.int32, 16)
                lane0 = iota == 0
                vi = ci_v[pl.ds(bi, 16)]
                ci_v[pl.ds(bi, 16)] = jnp.where(lane0, jnp.full((16,), li), vi)
                vx = cx_v[pl.ds(bi, 16)]
                cx_v[pl.ds(bi, 16)] = jnp.where(lane0, jnp.full((16,), lx), vx)
                vy = cy_v[pl.ds(bi, 16)]
                cy_v[pl.ds(bi, 16)] = jnp.where(lane0, jnp.full((16,), ly), vy)
                vz = cz_v[pl.ds(bi, 16)]
                cz_v[pl.ds(bi, 16)] = jnp.where(lane0, jnp.full((16,), lz), vz)
                return nk - 1

            lax.fori_loop(0, nk - K, remove_one, nk)

        for v in range(K // 16):
            fi_v[pl.ds(v * 16, 16)] = ci_v[pl.ds(v * 16, 16)]
        pltpu.async_copy(g_hbm.at[fi_v], grows_v, sem).wait()

        def pair(kp, acc):
            rxs = cx_v[pl.ds(kp, 16)][0]
            rys = cy_v[pl.ds(kp, 16)][0]
            rzs = cz_v[pl.ds(kp, 16)][0]
            out = []
            for c in range(8):
                gv = grows_v[kp, pl.ds(c * 16, 16)]
                h = (gv + rxs * wp_v[0, pl.ds(c * 16, 16)]
                     + rys * wp_v[1, pl.ds(c * 16, 16)]
                     + rzs * wp_v[2, pl.ds(c * 16, 16)])
                out.append(jnp.maximum(acc[c], jnp.maximum(h, 0.0)))
            return tuple(out)

        acc = lax.fori_loop(0, K, pair,
                            tuple(jnp.zeros((16,), jnp.float32)
                                  for _ in range(8)))
        obase = k * D
        for c in range(8):
            outq_v[pl.ds(obase + c * 16, 16)] = acc[c]
        return 0

    lax.fori_loop(0, QPW, per_query, 0)
    pltpu.sync_copy(outq_v, out_hbm.at[pl.ds(qbase * D, QPW * D)])


def _run_sc_conv(px, py, pz, qx, qy, qz, g, wp, cmins):
    mesh = plsc.VectorSubcoreMesh(core_axis_name="c", subcore_axis_name="s",
                                  num_cores=2, num_subcores=16)
    f = functools.partial(
        pl.kernel,
        out_type=jax.ShapeDtypeStruct((SPAD * D,), jnp.float32),
        mesh=mesh,
        scratch_types=[
            pltpu.VMEM((NPAD + 16,), jnp.float32),
            pltpu.VMEM((NPAD + 16,), jnp.float32),
            pltpu.VMEM((NPAD + 16,), jnp.float32),
            pltpu.VMEM((QPW + 16,), jnp.float32),
            pltpu.VMEM((QPW + 16,), jnp.float32),
            pltpu.VMEM((QPW + 16,), jnp.float32),
            pltpu.VMEM((CAP + 32,), jnp.int32),
            pltpu.VMEM((CAP + 32,), jnp.float32),
            pltpu.VMEM((CAP + 32,), jnp.float32),
            pltpu.VMEM((CAP + 32,), jnp.float32),
            pltpu.VMEM((96,), jnp.float32),
            pltpu.VMEM((640 + 16,), jnp.float32),
            pltpu.VMEM((K,), jnp.int32),
            pltpu.VMEM((K, D), jnp.float32),
            pltpu.VMEM((3, D), jnp.float32),
            pltpu.VMEM((QPW * D,), jnp.float32),
            pltpu.SemaphoreType.DMA,
        ],
    )(_sc_conv_body)
    return f(px, py, pz, qx, qy, qz, g, wp, cmins)


def kernel(x, pos, batch, W, b):
    wx = W[:D, :]
    wp = W[D:, :]
    b2 = b.reshape(1, D)
    x_pad = jnp.pad(x, ((0, NPAD - N), (0, 0)))
    g = _compute_g(x_pad, wx, b2)

    posp = jnp.pad(pos, ((0, NPAD - N), (0, 0)), constant_values=FAR)
    planes = posp.T.reshape(3, 8, 1280)
    idx_pad, selx, sely, selz = _run_fps(planes[0], planes[1], planes[2])

    pts = posp.reshape(640, 16, 3)
    cmins = _compute_cmins(selx, sely, selz,
                           pts[:, :, 0].T.copy(), pts[:, :, 1].T.copy(),
                           pts[:, :, 2].T.copy())

    out_flat = _run_sc_conv(
        planes[0].reshape(NPAD), planes[1].reshape(NPAD),
        planes[2].reshape(NPAD), selx, sely, selz, g, wp,
        cmins.reshape(SPAD * 640))

    out = out_flat.reshape(SPAD, D)[:S]
    idx = idx_pad[:S]
    pos_dst = jnp.stack([selx[:S], sely[:S], selz[:S]], axis=1)
    batch_dst = batch[idx]
    return (out, pos_dst, batch_dst)
